# Initial kernel scaffold; baseline (speedup 1.0000x reference)
import jax
import jax.numpy as jnp
from jax.experimental import pallas as pl


def kernel(x_ts, t_ts, global_means, W, b):
    raise NotImplementedError("write your pallas kernel here")


# speedup vs baseline: 41.0160x; 41.0160x over previous
import functools

import jax
import jax.numpy as jnp
from jax import lax
from jax.experimental import pallas as pl
from jax.experimental.pallas import tpu as pltpu
from jax.experimental.pallas import tpu_sc as plsc

D_M = 512
D_H = 64
ALPHA = 2048
L_OBS = 1024

_NC = 2
_NS = 16
_NW = _NC * _NS
_FPW = D_M // _NW
_LANES = 16
_CHUNKS = ALPHA // _LANES


def _impute_body(t_hbm, x_hbm, g_hbm, out_hbm, times_v, obs_v, row_v, g_v):
    wid = lax.axis_index("s") * _NC + lax.axis_index("c")
    f0 = wid * _FPW
    pltpu.sync_copy(g_hbm.at[pl.ds(f0, _FPW)], g_v)

    def per_feature(j, carry):
        f = f0 + j
        pltpu.sync_copy(t_hbm.at[f], times_v)
        pltpu.sync_copy(x_hbm.at[f], obs_v)
        gmean = plsc.load_gather(g_v, [jnp.full((_LANES,), j, jnp.int32)])

        def per_chunk(c, carry2):
            t_vec = lax.iota(jnp.int32, _LANES) + c * _LANES
            pos = jnp.zeros((_LANES,), jnp.int32)
            for step in (512, 256, 128, 64, 32, 16, 8, 4, 2, 1):
                v = plsc.load_gather(times_v, [pos + (step - 1)])
                pos = pos + jnp.where(v <= t_vec, step, 0)
            v = plsc.load_gather(times_v, [pos])
            pos = pos + (v <= t_vec).astype(jnp.int32)
            val = plsc.load_gather(obs_v, [jnp.maximum(pos - 1, 0)])
            row_v[pl.ds(c * _LANES, _LANES)] = jnp.where(pos > 0, val, gmean)
            return carry2

        lax.fori_loop(0, _CHUNKS, per_chunk, jnp.int32(0), unroll=2)
        pltpu.sync_copy(row_v, out_hbm.at[f])
        return carry

    lax.fori_loop(0, _FPW, per_feature, jnp.int32(0))


_impute_sc = functools.partial(
    pl.kernel,
    out_type=jax.ShapeDtypeStruct((D_M, ALPHA), jnp.float32),
    mesh=plsc.VectorSubcoreMesh(
        core_axis_name="c", subcore_axis_name="s",
        num_cores=_NC, num_subcores=_NS),
    compiler_params=pltpu.CompilerParams(needs_layout_passes=False),
    scratch_types=[
        pltpu.VMEM((L_OBS,), jnp.int32),
        pltpu.VMEM((L_OBS,), jnp.float32),
        pltpu.VMEM((ALPHA,), jnp.float32),
        pltpu.VMEM((_FPW,), jnp.float32),
    ],
)(_impute_body)


def _matmul_body(rs_ref, w_ref, b_ref, out_ref):
    out_ref[...] = lax.dot_general(
        rs_ref[...], w_ref[...], (((0,), (1,)), ((), ())),
        preferred_element_type=jnp.float32) + b_ref[...]


_A_BLK = 512


def _matmul_tc(rs, W, b2):
    return pl.pallas_call(
        _matmul_body,
        grid=(ALPHA // _A_BLK,),
        in_specs=[
            pl.BlockSpec((D_M, _A_BLK), lambda k: (0, k)),
            pl.BlockSpec((D_H, D_M), lambda k: (0, 0)),
            pl.BlockSpec((1, D_H), lambda k: (0, 0)),
        ],
        out_specs=pl.BlockSpec((_A_BLK, D_H), lambda k: (k, 0)),
        out_shape=jax.ShapeDtypeStruct((ALPHA, D_H), jnp.float32),
    )(rs, W, b2)


def kernel(x_ts, t_ts, global_means, W, b):
    t_ts = t_ts.astype(jnp.int32)
    rs = _impute_sc(t_ts, x_ts, global_means)
    return _matmul_tc(rs, W, b.reshape(1, D_H))


# speedup vs baseline: 126.8619x; 3.0930x over previous
import functools

import jax
import jax.numpy as jnp
from jax import lax
from jax.experimental import pallas as pl
from jax.experimental.pallas import tpu as pltpu
from jax.experimental.pallas import tpu_sc as plsc

D_M = 512
D_H = 64
ALPHA = 2048
L_OBS = 1024

_NC = 2
_NS = 16
_NW = _NC * _NS
_FPW = D_M // _NW
_LANES = 16
_CHUNKS = ALPHA // _LANES


def _impute_body(t_hbm, x_hbm, g_hbm, out_hbm, times_v, obs_v, row_v, g_v):
    wid = lax.axis_index("s") * _NC + lax.axis_index("c")
    f0 = wid * _FPW
    pltpu.sync_copy(g_hbm.at[pl.ds(f0, _FPW)], g_v)

    def per_feature(j, carry):
        f = f0 + j
        pltpu.sync_copy(t_hbm.at[f], times_v)
        pltpu.sync_copy(x_hbm.at[f], obs_v)
        gmean = plsc.load_gather(g_v, [jnp.full((_LANES,), j, jnp.int32)])

        @plsc.parallel_loop(0, _CHUNKS, unroll=4)
        def per_chunk(c):
            t_vec = lax.iota(jnp.int32, _LANES) + c * _LANES
            pos = jnp.zeros((_LANES,), jnp.int32)
            for step in (512, 256, 128, 64, 32, 16, 8, 4, 2, 1):
                v = plsc.load_gather(times_v, [pos + (step - 1)])
                pos = pos + jnp.where(v <= t_vec, step, 0)
            v = plsc.load_gather(times_v, [pos])
            pos = pos + (v <= t_vec).astype(jnp.int32)
            val = plsc.load_gather(obs_v, [jnp.maximum(pos - 1, 0)])
            row_v[pl.ds(c * _LANES, _LANES)] = jnp.where(pos > 0, val, gmean)
        pltpu.sync_copy(row_v, out_hbm.at[f])
        return carry

    lax.fori_loop(0, _FPW, per_feature, jnp.int32(0))


_impute_sc = functools.partial(
    pl.kernel,
    out_type=jax.ShapeDtypeStruct((D_M, ALPHA), jnp.float32),
    mesh=plsc.VectorSubcoreMesh(
        core_axis_name="c", subcore_axis_name="s",
        num_cores=_NC, num_subcores=_NS),
    compiler_params=pltpu.CompilerParams(needs_layout_passes=False),
    scratch_types=[
        pltpu.VMEM((L_OBS,), jnp.int32),
        pltpu.VMEM((L_OBS,), jnp.float32),
        pltpu.VMEM((ALPHA,), jnp.float32),
        pltpu.VMEM((_FPW,), jnp.float32),
    ],
)(_impute_body)


def _matmul_body(rs_ref, w_ref, b_ref, out_ref):
    out_ref[...] = lax.dot_general(
        rs_ref[...], w_ref[...], (((0,), (1,)), ((), ())),
        preferred_element_type=jnp.float32) + b_ref[...]


_A_BLK = 512


def _matmul_tc(rs, W, b2):
    return pl.pallas_call(
        _matmul_body,
        grid=(ALPHA // _A_BLK,),
        in_specs=[
            pl.BlockSpec((D_M, _A_BLK), lambda k: (0, k)),
            pl.BlockSpec((D_H, D_M), lambda k: (0, 0)),
            pl.BlockSpec((1, D_H), lambda k: (0, 0)),
        ],
        out_specs=pl.BlockSpec((_A_BLK, D_H), lambda k: (k, 0)),
        out_shape=jax.ShapeDtypeStruct((ALPHA, D_H), jnp.float32),
    )(rs, W, b2)


def kernel(x_ts, t_ts, global_means, W, b):
    t_ts = t_ts.astype(jnp.int32)
    rs = _impute_sc(t_ts, x_ts, global_means)
    return _matmul_tc(rs, W, b.reshape(1, D_H))
